# baseline (device time: 50770 ns/iter reference)
import functools

import jax
import jax.numpy as jnp
from jax import lax
from jax.experimental import pallas as pl
from jax.experimental.pallas import tpu as pltpu

N_DEV = 4
N_HALF = 2
N_WSPLIT = 4

ORDER = (2, 1, 3, 0)


def kernel(x, w_mat):
    m_per, k_dim = x.shape
    n_total = w_mat.shape[1]
    n_per = n_total // N_DEV
    m_total = N_DEV * m_per
    m_half = m_per // N_HALF

    def body(x_hbm, w_hbm, out_ref, x_v, w_v, sq, ssc, rq, rsc,
             x_sems, w_sems, sq_sems, ssc_sems, rq_sems, rsc_sems):
        my = lax.axis_index("i")

        x_copies = []
        for h in range(N_HALF):
            c = pltpu.make_async_copy(
                x_hbm.at[pl.ds(h * m_half, m_half), :],
                x_v.at[pl.ds(h * m_half, m_half), :],
                x_sems.at[h],
            )
            c.start()
            x_copies.append(c)

        w_copies = []
        k_quarter = k_dim // N_WSPLIT
        for idx, k in enumerate(ORDER):
            tgt = (my + k) % N_DEV
            qs = []
            for q in range(N_WSPLIT):
                c = pltpu.make_async_copy(
                    w_hbm.at[pl.ds(q * k_quarter, k_quarter),
                             pl.ds(tgt * n_per, n_per)],
                    w_v.at[idx, pl.ds(q * k_quarter, k_quarter), :],
                    w_sems.at[idx, q],
                )
                c.start()
                qs.append(c)
            w_copies.append(qs)

        barrier = pltpu.get_barrier_semaphore()
        for k in range(1, N_DEV):
            peer = (my + k) % N_DEV
            pl.semaphore_signal(
                barrier, inc=1,
                device_id=(peer,), device_id_type=pl.DeviceIdType.MESH,
            )
        pl.semaphore_wait(barrier, N_DEV - 1)

        sends = []
        for h in range(N_HALF):
            x_copies[h].wait()
            for idx, k in enumerate(ORDER):
                tgt = (my + k) % N_DEV
                if h == 0:
                    for c in w_copies[idx]:
                        c.wait()
                block = jnp.dot(
                    x_v[pl.ds(h * m_half, m_half), :], w_v[idx, :, :],
                    preferred_element_type=jnp.float32,
                )
                if k == 0:
                    out_ref[pl.ds(my * m_per + h * m_half, m_half), :] = block
                else:
                    slot = idx * N_HALF + h
                    m = jnp.max(jnp.abs(block))
                    inv = 127.0 / jnp.maximum(m, 1e-20)
                    sq[slot, :, :] = jnp.clip(
                        jnp.round(block * inv), -127.0, 127.0
                    ).astype(jnp.int8)
                    ssc[slot, :, :] = jnp.full(
                        (8, 128), m * (1.0 / 127.0), jnp.float32
                    )
                    rdma_q = pltpu.make_async_remote_copy(
                        src_ref=sq.at[slot],
                        dst_ref=rq.at[my, h],
                        send_sem=sq_sems.at[slot],
                        recv_sem=rq_sems.at[my, h],
                        device_id=(tgt,),
                        device_id_type=pl.DeviceIdType.MESH,
                    )
                    rdma_q.start()
                    rdma_s = pltpu.make_async_remote_copy(
                        src_ref=ssc.at[slot],
                        dst_ref=rsc.at[my, h],
                        send_sem=ssc_sems.at[slot],
                        recv_sem=rsc_sems.at[my, h],
                        device_id=(tgt,),
                        device_id_type=pl.DeviceIdType.MESH,
                    )
                    rdma_s.start()
                    sends.append(rdma_q)
                    sends.append(rdma_s)

        for h in range(N_HALF):
            for k in ORDER[:-1]:
                src = (my + k) % N_DEV
                recv_q = pltpu.make_async_remote_copy(
                    src_ref=sq.at[0],
                    dst_ref=rq.at[src, h],
                    send_sem=sq_sems.at[0],
                    recv_sem=rq_sems.at[src, h],
                    device_id=(src,),
                    device_id_type=pl.DeviceIdType.MESH,
                )
                recv_q.wait_recv()
                recv_s = pltpu.make_async_remote_copy(
                    src_ref=ssc.at[0],
                    dst_ref=rsc.at[src, h],
                    send_sem=ssc_sems.at[0],
                    recv_sem=rsc_sems.at[src, h],
                    device_id=(src,),
                    device_id_type=pl.DeviceIdType.MESH,
                )
                recv_s.wait_recv()
                scale = jnp.max(rsc[src, h, :, :])
                out_ref[pl.ds(src * m_per + h * m_half, m_half), :] = (
                    rq[src, h, :, :].astype(jnp.float32) * scale
                )
        for rdma in sends:
            rdma.wait_send()

        @functools.partial(
            pl.run_scoped, exit_sem=pltpu.SemaphoreType.REGULAR
        )
        def _(exit_sem):
            for k in range(1, N_DEV):
                peer = (my + k) % N_DEV
                pl.semaphore_signal(
                    exit_sem, inc=1,
                    device_id=(peer,), device_id_type=pl.DeviceIdType.MESH,
                )
            pl.semaphore_wait(exit_sem, N_DEV - 1)

    n_slots = (N_DEV - 1) * N_HALF
    return pl.pallas_call(
        body,
        out_shape=jax.ShapeDtypeStruct((m_total, n_per), jnp.float32),
        in_specs=[
            pl.BlockSpec(memory_space=pltpu.MemorySpace.HBM),
            pl.BlockSpec(memory_space=pltpu.MemorySpace.HBM),
        ],
        out_specs=pl.BlockSpec(memory_space=pltpu.VMEM),
        scratch_shapes=[
            pltpu.VMEM((m_per, k_dim), jnp.float32),
            pltpu.VMEM((N_DEV, k_dim, n_per), jnp.float32),
            pltpu.VMEM((n_slots, m_half, n_per), jnp.int8),
            pltpu.VMEM((n_slots, 8, 128), jnp.float32),
            pltpu.VMEM((N_DEV, N_HALF, m_half, n_per), jnp.int8),
            pltpu.VMEM((N_DEV, N_HALF, 8, 128), jnp.float32),
            pltpu.SemaphoreType.DMA((N_HALF,)),
            pltpu.SemaphoreType.DMA((N_DEV, N_WSPLIT)),
            pltpu.SemaphoreType.DMA((n_slots,)),
            pltpu.SemaphoreType.DMA((n_slots,)),
            pltpu.SemaphoreType.DMA((N_DEV, N_HALF)),
            pltpu.SemaphoreType.DMA((N_DEV, N_HALF)),
        ],
        compiler_params=pltpu.CompilerParams(
            collective_id=0,
            vmem_limit_bytes=62 * 1024 * 1024,
        ),
    )(x, w_mat)


# device time: 47259 ns/iter; 1.0743x vs baseline; 1.0743x over previous
import functools

import jax
import jax.numpy as jnp
from jax import lax
from jax.experimental import pallas as pl
from jax.experimental.pallas import tpu as pltpu

N_DEV = 4
N_HALF = 2

ORDER = (2, 1, 3, 0)


def kernel(x, w_mat):
    m_per, k_dim = x.shape
    n_total = w_mat.shape[1]
    n_per = n_total // N_DEV
    m_total = N_DEV * m_per
    m_half = m_per // N_HALF

    def body(x_hbm, w_hbm, out_ref, x_v, w_v, sq, ssc, rq, rsc,
             x_sems, w_sems, sq_sems, ssc_sems, rq_sems, rsc_sems):
        my = lax.axis_index("i")

        x_copies = []
        for h in range(N_HALF):
            c = pltpu.make_async_copy(
                x_hbm.at[pl.ds(h * m_half, m_half), :],
                x_v.at[pl.ds(h * m_half, m_half), :],
                x_sems.at[h],
            )
            c.start()
            x_copies.append(c)

        w_copies = []
        for idx, k in enumerate(ORDER):
            tgt = (my + k) % N_DEV
            c = pltpu.make_async_copy(
                w_hbm.at[:, pl.ds(tgt * n_per, n_per)],
                w_v.at[idx],
                w_sems.at[idx],
            )
            c.start()
            w_copies.append(c)

        barrier = pltpu.get_barrier_semaphore()
        for k in range(1, N_DEV):
            peer = (my + k) % N_DEV
            pl.semaphore_signal(
                barrier, inc=1,
                device_id=(peer,), device_id_type=pl.DeviceIdType.MESH,
            )
        pl.semaphore_wait(barrier, N_DEV - 1)

        sends = []
        for h in range(N_HALF):
            x_copies[h].wait()
            for idx, k in enumerate(ORDER):
                tgt = (my + k) % N_DEV
                if h == 0:
                    w_copies[idx].wait()
                block = jnp.dot(
                    x_v[pl.ds(h * m_half, m_half), :], w_v[idx, :, :],
                    preferred_element_type=jnp.float32,
                )
                if k == 0:
                    out_ref[pl.ds(my * m_per + h * m_half, m_half), :] = block
                else:
                    slot = idx * N_HALF + h
                    m = jnp.max(jnp.abs(block))
                    inv = 127.0 / jnp.maximum(m, 1e-20)
                    sq[slot, :, :] = jnp.clip(
                        jnp.round(block * inv), -127.0, 127.0
                    ).astype(jnp.int8)
                    ssc[slot, :, :] = jnp.full(
                        (8, 128), m * (1.0 / 127.0), jnp.float32
                    )
                    rdma_q = pltpu.make_async_remote_copy(
                        src_ref=sq.at[slot],
                        dst_ref=rq.at[my, h],
                        send_sem=sq_sems.at[slot],
                        recv_sem=rq_sems.at[my, h],
                        device_id=(tgt,),
                        device_id_type=pl.DeviceIdType.MESH,
                    )
                    rdma_q.start()
                    rdma_s = pltpu.make_async_remote_copy(
                        src_ref=ssc.at[slot],
                        dst_ref=rsc.at[my, h],
                        send_sem=ssc_sems.at[slot],
                        recv_sem=rsc_sems.at[my, h],
                        device_id=(tgt,),
                        device_id_type=pl.DeviceIdType.MESH,
                    )
                    rdma_s.start()
                    sends.append(rdma_q)
                    sends.append(rdma_s)

        for h in range(N_HALF):
            for k in ORDER[:-1]:
                src = (my + k) % N_DEV
                recv_q = pltpu.make_async_remote_copy(
                    src_ref=sq.at[0],
                    dst_ref=rq.at[src, h],
                    send_sem=sq_sems.at[0],
                    recv_sem=rq_sems.at[src, h],
                    device_id=(src,),
                    device_id_type=pl.DeviceIdType.MESH,
                )
                recv_q.wait_recv()
                recv_s = pltpu.make_async_remote_copy(
                    src_ref=ssc.at[0],
                    dst_ref=rsc.at[src, h],
                    send_sem=ssc_sems.at[0],
                    recv_sem=rsc_sems.at[src, h],
                    device_id=(src,),
                    device_id_type=pl.DeviceIdType.MESH,
                )
                recv_s.wait_recv()
                scale = jnp.max(rsc[src, h, :, :])
                out_ref[pl.ds(src * m_per + h * m_half, m_half), :] = (
                    rq[src, h, :, :].astype(jnp.float32) * scale
                )
        for rdma in sends:
            rdma.wait_send()

        @functools.partial(
            pl.run_scoped, exit_sem=pltpu.SemaphoreType.REGULAR
        )
        def _(exit_sem):
            for k in range(1, N_DEV):
                peer = (my + k) % N_DEV
                pl.semaphore_signal(
                    exit_sem, inc=1,
                    device_id=(peer,), device_id_type=pl.DeviceIdType.MESH,
                )
            pl.semaphore_wait(exit_sem, N_DEV - 1)

    n_slots = (N_DEV - 1) * N_HALF
    return pl.pallas_call(
        body,
        out_shape=jax.ShapeDtypeStruct((m_total, n_per), jnp.float32),
        in_specs=[
            pl.BlockSpec(memory_space=pltpu.MemorySpace.HBM),
            pl.BlockSpec(memory_space=pltpu.MemorySpace.HBM),
        ],
        out_specs=pl.BlockSpec(memory_space=pltpu.VMEM),
        scratch_shapes=[
            pltpu.VMEM((m_per, k_dim), jnp.float32),
            pltpu.VMEM((N_DEV, k_dim, n_per), jnp.float32),
            pltpu.VMEM((n_slots, m_half, n_per), jnp.int8),
            pltpu.VMEM((n_slots, 8, 128), jnp.float32),
            pltpu.VMEM((N_DEV, N_HALF, m_half, n_per), jnp.int8),
            pltpu.VMEM((N_DEV, N_HALF, 8, 128), jnp.float32),
            pltpu.SemaphoreType.DMA((N_HALF,)),
            pltpu.SemaphoreType.DMA((N_DEV,)),
            pltpu.SemaphoreType.DMA((n_slots,)),
            pltpu.SemaphoreType.DMA((n_slots,)),
            pltpu.SemaphoreType.DMA((N_DEV, N_HALF)),
            pltpu.SemaphoreType.DMA((N_DEV, N_HALF)),
        ],
        compiler_params=pltpu.CompilerParams(
            collective_id=0,
            vmem_limit_bytes=62 * 1024 * 1024,
        ),
    )(x, w_mat)


# device time: 45882 ns/iter; 1.1065x vs baseline; 1.0300x over previous
import jax
import jax.numpy as jnp
from jax import lax
from jax.experimental import pallas as pl
from jax.experimental.pallas import tpu as pltpu

N_DEV = 4
N_HALF = 2

ORDER = (2, 1, 3, 0)


def kernel(x, w_mat):
    m_per, k_dim = x.shape
    n_total = w_mat.shape[1]
    n_per = n_total // N_DEV
    m_total = N_DEV * m_per
    m_half = m_per // N_HALF

    def body(x_hbm, w_hbm, out_ref, x_v, w_v, sq, ssc, rq, rsc,
             x_sems, w_sems, sq_sems, ssc_sems, rq_sems, rsc_sems):
        my = lax.axis_index("i")

        x_copies = []
        for h in range(N_HALF):
            c = pltpu.make_async_copy(
                x_hbm.at[pl.ds(h * m_half, m_half), :],
                x_v.at[pl.ds(h * m_half, m_half), :],
                x_sems.at[h],
            )
            c.start()
            x_copies.append(c)

        w_copies = []
        for idx, k in enumerate(ORDER):
            tgt = (my + k) % N_DEV
            c = pltpu.make_async_copy(
                w_hbm.at[:, pl.ds(tgt * n_per, n_per)],
                w_v.at[idx],
                w_sems.at[idx],
            )
            c.start()
            w_copies.append(c)

        barrier = pltpu.get_barrier_semaphore()
        for k in range(1, N_DEV):
            peer = (my + k) % N_DEV
            pl.semaphore_signal(
                barrier, inc=1,
                device_id=(peer,), device_id_type=pl.DeviceIdType.MESH,
            )
        pl.semaphore_wait(barrier, N_DEV - 1)

        sends = []
        for h in range(N_HALF):
            x_copies[h].wait()
            for idx, k in enumerate(ORDER):
                tgt = (my + k) % N_DEV
                if h == 0:
                    w_copies[idx].wait()
                block = jnp.dot(
                    x_v[pl.ds(h * m_half, m_half), :], w_v[idx, :, :],
                    preferred_element_type=jnp.float32,
                )
                if k == 0:
                    out_ref[pl.ds(my * m_per + h * m_half, m_half), :] = block
                else:
                    slot = idx * N_HALF + h
                    m = jnp.max(jnp.abs(block))
                    inv = 127.0 / jnp.maximum(m, 1e-20)
                    sq[slot, :, :] = jnp.clip(
                        jnp.round(block * inv), -127.0, 127.0
                    ).astype(jnp.int8)
                    ssc[slot, :, :] = jnp.full(
                        (8, 128), m * (1.0 / 127.0), jnp.float32
                    )
                    rdma_q = pltpu.make_async_remote_copy(
                        src_ref=sq.at[slot],
                        dst_ref=rq.at[my, h],
                        send_sem=sq_sems.at[slot],
                        recv_sem=rq_sems.at[my, h],
                        device_id=(tgt,),
                        device_id_type=pl.DeviceIdType.MESH,
                    )
                    rdma_q.start()
                    rdma_s = pltpu.make_async_remote_copy(
                        src_ref=ssc.at[slot],
                        dst_ref=rsc.at[my, h],
                        send_sem=ssc_sems.at[slot],
                        recv_sem=rsc_sems.at[my, h],
                        device_id=(tgt,),
                        device_id_type=pl.DeviceIdType.MESH,
                    )
                    rdma_s.start()
                    sends.append(rdma_q)
                    sends.append(rdma_s)

        for h in range(N_HALF):
            for k in ORDER[:-1]:
                src = (my + k) % N_DEV
                recv_q = pltpu.make_async_remote_copy(
                    src_ref=sq.at[0],
                    dst_ref=rq.at[src, h],
                    send_sem=sq_sems.at[0],
                    recv_sem=rq_sems.at[src, h],
                    device_id=(src,),
                    device_id_type=pl.DeviceIdType.MESH,
                )
                recv_q.wait_recv()
                recv_s = pltpu.make_async_remote_copy(
                    src_ref=ssc.at[0],
                    dst_ref=rsc.at[src, h],
                    send_sem=ssc_sems.at[0],
                    recv_sem=rsc_sems.at[src, h],
                    device_id=(src,),
                    device_id_type=pl.DeviceIdType.MESH,
                )
                recv_s.wait_recv()
                scale = jnp.max(rsc[src, h, :, :])
                out_ref[pl.ds(src * m_per + h * m_half, m_half), :] = (
                    rq[src, h, :, :].astype(jnp.float32) * scale
                )
        for rdma in sends:
            rdma.wait_send()


    n_slots = (N_DEV - 1) * N_HALF
    return pl.pallas_call(
        body,
        out_shape=jax.ShapeDtypeStruct((m_total, n_per), jnp.float32),
        in_specs=[
            pl.BlockSpec(memory_space=pltpu.MemorySpace.HBM),
            pl.BlockSpec(memory_space=pltpu.MemorySpace.HBM),
        ],
        out_specs=pl.BlockSpec(memory_space=pltpu.VMEM),
        scratch_shapes=[
            pltpu.VMEM((m_per, k_dim), jnp.float32),
            pltpu.VMEM((N_DEV, k_dim, n_per), jnp.float32),
            pltpu.VMEM((n_slots, m_half, n_per), jnp.int8),
            pltpu.VMEM((n_slots, 8, 128), jnp.float32),
            pltpu.VMEM((N_DEV, N_HALF, m_half, n_per), jnp.int8),
            pltpu.VMEM((N_DEV, N_HALF, 8, 128), jnp.float32),
            pltpu.SemaphoreType.DMA((N_HALF,)),
            pltpu.SemaphoreType.DMA((N_DEV,)),
            pltpu.SemaphoreType.DMA((n_slots,)),
            pltpu.SemaphoreType.DMA((n_slots,)),
            pltpu.SemaphoreType.DMA((N_DEV, N_HALF)),
            pltpu.SemaphoreType.DMA((N_DEV, N_HALF)),
        ],
        compiler_params=pltpu.CompilerParams(
            collective_id=0,
            vmem_limit_bytes=62 * 1024 * 1024,
        ),
    )(x, w_mat)


# device time: 44803 ns/iter; 1.1332x vs baseline; 1.0241x over previous
import jax
import jax.numpy as jnp
from jax import lax
from jax.experimental import pallas as pl
from jax.experimental.pallas import tpu as pltpu

N_DEV = 4
N_HALF = 1

ORDER = (2, 1, 3, 0)


def kernel(x, w_mat):
    m_per, k_dim = x.shape
    n_total = w_mat.shape[1]
    n_per = n_total // N_DEV
    m_total = N_DEV * m_per
    m_half = m_per // N_HALF

    def body(x_hbm, w_hbm, out_ref, x_v, w_v, sq, ssc, rq, rsc,
             x_sems, w_sems, sq_sems, ssc_sems, rq_sems, rsc_sems):
        my = lax.axis_index("i")

        x_copies = []
        for h in range(N_HALF):
            c = pltpu.make_async_copy(
                x_hbm.at[pl.ds(h * m_half, m_half), :],
                x_v.at[pl.ds(h * m_half, m_half), :],
                x_sems.at[h],
            )
            c.start()
            x_copies.append(c)

        w_copies = []
        for idx, k in enumerate(ORDER):
            tgt = (my + k) % N_DEV
            c = pltpu.make_async_copy(
                w_hbm.at[:, pl.ds(tgt * n_per, n_per)],
                w_v.at[idx],
                w_sems.at[idx],
            )
            c.start()
            w_copies.append(c)

        barrier = pltpu.get_barrier_semaphore()
        for k in range(1, N_DEV):
            peer = (my + k) % N_DEV
            pl.semaphore_signal(
                barrier, inc=1,
                device_id=(peer,), device_id_type=pl.DeviceIdType.MESH,
            )
        pl.semaphore_wait(barrier, N_DEV - 1)

        sends = []
        for h in range(N_HALF):
            x_copies[h].wait()
            for idx, k in enumerate(ORDER):
                tgt = (my + k) % N_DEV
                if h == 0:
                    w_copies[idx].wait()
                block = jnp.dot(
                    x_v[pl.ds(h * m_half, m_half), :], w_v[idx, :, :],
                    preferred_element_type=jnp.float32,
                )
                if k == 0:
                    out_ref[pl.ds(my * m_per + h * m_half, m_half), :] = block
                else:
                    slot = idx * N_HALF + h
                    m = jnp.max(jnp.abs(block))
                    inv = 127.0 / jnp.maximum(m, 1e-20)
                    sq[slot, :, :] = jnp.clip(
                        jnp.round(block * inv), -127.0, 127.0
                    ).astype(jnp.int8)
                    ssc[slot, :, :] = jnp.full(
                        (8, 128), m * (1.0 / 127.0), jnp.float32
                    )
                    rdma_q = pltpu.make_async_remote_copy(
                        src_ref=sq.at[slot],
                        dst_ref=rq.at[my, h],
                        send_sem=sq_sems.at[slot],
                        recv_sem=rq_sems.at[my, h],
                        device_id=(tgt,),
                        device_id_type=pl.DeviceIdType.MESH,
                    )
                    rdma_q.start()
                    rdma_s = pltpu.make_async_remote_copy(
                        src_ref=ssc.at[slot],
                        dst_ref=rsc.at[my, h],
                        send_sem=ssc_sems.at[slot],
                        recv_sem=rsc_sems.at[my, h],
                        device_id=(tgt,),
                        device_id_type=pl.DeviceIdType.MESH,
                    )
                    rdma_s.start()
                    sends.append(rdma_q)
                    sends.append(rdma_s)

        for h in range(N_HALF):
            for k in ORDER[:-1]:
                src = (my + k) % N_DEV
                recv_q = pltpu.make_async_remote_copy(
                    src_ref=sq.at[0],
                    dst_ref=rq.at[src, h],
                    send_sem=sq_sems.at[0],
                    recv_sem=rq_sems.at[src, h],
                    device_id=(src,),
                    device_id_type=pl.DeviceIdType.MESH,
                )
                recv_q.wait_recv()
                recv_s = pltpu.make_async_remote_copy(
                    src_ref=ssc.at[0],
                    dst_ref=rsc.at[src, h],
                    send_sem=ssc_sems.at[0],
                    recv_sem=rsc_sems.at[src, h],
                    device_id=(src,),
                    device_id_type=pl.DeviceIdType.MESH,
                )
                recv_s.wait_recv()
                scale = jnp.max(rsc[src, h, :, :])
                out_ref[pl.ds(src * m_per + h * m_half, m_half), :] = (
                    rq[src, h, :, :].astype(jnp.float32) * scale
                )
        for rdma in sends:
            rdma.wait_send()


    n_slots = (N_DEV - 1) * N_HALF
    return pl.pallas_call(
        body,
        out_shape=jax.ShapeDtypeStruct((m_total, n_per), jnp.float32),
        in_specs=[
            pl.BlockSpec(memory_space=pltpu.MemorySpace.HBM),
            pl.BlockSpec(memory_space=pltpu.MemorySpace.HBM),
        ],
        out_specs=pl.BlockSpec(memory_space=pltpu.VMEM),
        scratch_shapes=[
            pltpu.VMEM((m_per, k_dim), jnp.float32),
            pltpu.VMEM((N_DEV, k_dim, n_per), jnp.float32),
            pltpu.VMEM((n_slots, m_half, n_per), jnp.int8),
            pltpu.VMEM((n_slots, 8, 128), jnp.float32),
            pltpu.VMEM((N_DEV, N_HALF, m_half, n_per), jnp.int8),
            pltpu.VMEM((N_DEV, N_HALF, 8, 128), jnp.float32),
            pltpu.SemaphoreType.DMA((N_HALF,)),
            pltpu.SemaphoreType.DMA((N_DEV,)),
            pltpu.SemaphoreType.DMA((n_slots,)),
            pltpu.SemaphoreType.DMA((n_slots,)),
            pltpu.SemaphoreType.DMA((N_DEV, N_HALF)),
            pltpu.SemaphoreType.DMA((N_DEV, N_HALF)),
        ],
        compiler_params=pltpu.CompilerParams(
            collective_id=0,
            vmem_limit_bytes=62 * 1024 * 1024,
        ),
    )(x, w_mat)
